# Initial kernel scaffold; baseline (speedup 1.0000x reference)
import jax
import jax.numpy as jnp
from jax.experimental import pallas as pl


def kernel(x, pos, vel, edge_index, params):
    raise NotImplementedError("write your pallas kernel here")


# speedup vs baseline: 4.7967x; 4.7967x over previous
import functools

import jax
import jax.numpy as jnp
from jax import lax
from jax.experimental import pallas as pl
from jax.experimental.pallas import tpu as pltpu
from jax.experimental.pallas import tpu_sc as plsc

NC = 2
NS = 16
NW = NC * NS
LANES = 16

N = 10000
E = 160000
D = 256
H = 64

TW = 256
MW = 128
EPT = E // NW
BG = 40
NBG = EPT // BG
BS = 40
NBS = EPT // BS
NP = 10240
NPT = NP // NS

@functools.lru_cache(maxsize=None)
def _sc_mesh():
    return plsc.VectorSubcoreMesh(
        core_axis_name="c", subcore_axis_name="s",
        num_cores=NC, num_subcores=NS)


def _silu(t):
    return t / (1.0 + jnp.exp(-t))


def _proj_body(x_ref, ws_ref, wd_ref, b1_ref, g_ref, ts_ref, td_ref):
    xb = x_ref[...]
    gb = g_ref[...]
    ps = jnp.dot(xb, ws_ref[...], preferred_element_type=jnp.float32)
    pd = jnp.dot(xb, wd_ref[...], preferred_element_type=jnp.float32)
    pad = jnp.zeros((xb.shape[0], TW - 144), jnp.float32)
    ts_ref[...] = jnp.concatenate([ps, gb, pad], axis=1)
    td_ref[...] = jnp.concatenate([pd + b1_ref[...], -gb, pad], axis=1)


def _node_proj(x, wsrc, wdst, b1, geom):
    nb = 1000
    grid = N // nb
    return pl.pallas_call(
        _proj_body,
        grid=(grid,),
        in_specs=[
            pl.BlockSpec((nb, D), lambda i: (i, 0)),
            pl.BlockSpec((D, 128), lambda i: (0, 0)),
            pl.BlockSpec((D, 128), lambda i: (0, 0)),
            pl.BlockSpec((1, 128), lambda i: (0, 0)),
            pl.BlockSpec((nb, 16), lambda i: (i, 0)),
        ],
        out_specs=[
            pl.BlockSpec((nb, TW), lambda i: (i, 0)),
            pl.BlockSpec((nb, TW), lambda i: (i, 0)),
        ],
        out_shape=[
            jax.ShapeDtypeStruct((N, TW), jnp.float32),
            jax.ShapeDtypeStruct((N, TW), jnp.float32),
        ],
    )(x, wsrc, wdst, b1, geom)


@functools.lru_cache(maxsize=None)
def _edge_gather_kernel():
    return pl.kernel(
        _edge_gather_body,
        out_type=jax.ShapeDtypeStruct((E, TW), jnp.float32),
        mesh=_sc_mesh(),
        scratch_types=[
            pltpu.VMEM((EPT,), jnp.int32),
            pltpu.VMEM((EPT,), jnp.int32),
            pltpu.VMEM((BG, TW), jnp.float32),
            pltpu.VMEM((BG, TW), jnp.float32),
            pltpu.SemaphoreType.DMA,
            pltpu.SemaphoreType.DMA,
        ],
    )


def _edge_gather_body(ts_hbm, td_hbm, src_hbm, dst_hbm, out_hbm,
                      src_v, dst_v, bufs, bufd, sem1, sem2):
    c = lax.axis_index("c")
    s = lax.axis_index("s")
    wid = s * NC + c
    base = pl.multiple_of(wid * EPT, 8)
    pltpu.sync_copy(src_hbm.at[pl.ds(base, EPT)], src_v)
    pltpu.sync_copy(dst_hbm.at[pl.ds(base, EPT)], dst_v)

    def blk(j, carry):
        o = pl.multiple_of(j * BG, 8)
        cp1 = pltpu.async_copy(ts_hbm.at[src_v.at[pl.ds(o, BG)]], bufs, sem1)
        cp2 = pltpu.async_copy(td_hbm.at[dst_v.at[pl.ds(o, BG)]], bufd, sem2)
        cp1.wait()
        cp2.wait()

        def row(i, carry2):
            for k2 in range(144 // LANES):
                sl = (i, pl.ds(k2 * LANES, LANES))
                bufs[sl] = bufs[sl] + bufd[sl]
            return carry2

        lax.fori_loop(0, BG, row, 0)
        pltpu.sync_copy(bufs, out_hbm.at[pl.ds(base + o, BG)])
        return carry

    lax.fori_loop(0, NBG, blk, 0)


def _edge_mlp_body(r_ref, we2_ref, be2_ref, we3_ref, be3_ref,
                   wv2_ref, bv2_ref, r512_ref, r513_ref, msg_ref):
    rb = r_ref[...]
    relp = rb[:, 128:130]
    relv = rb[:, 130:132]
    ds = jnp.sum(relp * relp, axis=1, keepdims=True)
    dv = jnp.sum(relv * relp, axis=1, keepdims=True)
    t = rb[:, 0:128] + ds * r512_ref[...] + dv * r513_ref[...]
    e1 = _silu(t[:, 0:H])
    v1 = _silu(t[:, H:2 * H])
    h2 = _silu(jnp.dot(e1, we2_ref[...], preferred_element_type=jnp.float32)
               + be2_ref[...])
    mh = jnp.dot(h2, we3_ref[...], preferred_element_type=jnp.float32) + be3_ref[...]
    vw = jnp.dot(v1, wv2_ref[...], preferred_element_type=jnp.float32) + bv2_ref[...]
    lane = lax.broadcasted_iota(jnp.int32, (1, 16), 1)
    mask = (lane < 2).astype(jnp.float32)
    mv = vw * rb[:, 128:144] * mask
    padm = jnp.zeros((rb.shape[0], MW - H - 16), jnp.float32)
    msg_ref[...] = jnp.concatenate([mh, mv, padm], axis=1)


def _edge_mlp(r, we2, be2, we3, be3, wv2, bv2, r512, r513):
    eb = 2000
    grid = E // eb
    return pl.pallas_call(
        _edge_mlp_body,
        grid=(grid,),
        in_specs=[
            pl.BlockSpec((eb, TW), lambda i: (i, 0)),
            pl.BlockSpec((H, H), lambda i: (0, 0)),
            pl.BlockSpec((1, H), lambda i: (0, 0)),
            pl.BlockSpec((H, H), lambda i: (0, 0)),
            pl.BlockSpec((1, H), lambda i: (0, 0)),
            pl.BlockSpec((H, 1), lambda i: (0, 0)),
            pl.BlockSpec((1, 1), lambda i: (0, 0)),
            pl.BlockSpec((1, 128), lambda i: (0, 0)),
            pl.BlockSpec((1, 128), lambda i: (0, 0)),
        ],
        out_specs=pl.BlockSpec((eb, MW), lambda i: (i, 0)),
        out_shape=jax.ShapeDtypeStruct((E, MW), jnp.float32),
    )(r, we2, be2, we3, be3, wv2, bv2, r512, r513)


@functools.lru_cache(maxsize=None)
def _edge_scatter_kernel():
    return pl.kernel(
        _edge_scatter_body,
        out_type=jax.ShapeDtypeStruct((NC, NP, MW), jnp.float32),
        mesh=_sc_mesh(),
        scratch_types=[
            pltpu.VMEM((BS,), jnp.int32),
            pltpu.VMEM((BS, MW), jnp.float32),
            pltpu.VMEM_SHARED((NP, MW), jnp.float32),
        ],
    )


def _edge_scatter_body(msg_hbm, dsti_hbm, zeros_hbm, out_hbm, idx_blk, mbuf, acc):
    c = lax.axis_index("c")
    s = lax.axis_index("s")
    wid = s * NC + c
    rows0 = pl.multiple_of(s * NPT, 8)
    pltpu.sync_copy(zeros_hbm.at[pl.ds(rows0, NPT)], acc.at[pl.ds(rows0, NPT)])
    plsc.subcore_barrier()

    def blk(j, carry):
        base = pl.multiple_of(wid * EPT + j * BS, 8)
        pltpu.sync_copy(dsti_hbm.at[pl.ds(base, BS)], idx_blk)
        pltpu.sync_copy(msg_hbm.at[pl.ds(base, BS)], mbuf)
        pltpu.sync_copy(mbuf, acc.at[idx_blk], add=True)
        return carry

    lax.fori_loop(0, NBS, blk, 0)
    plsc.subcore_barrier()
    pltpu.sync_copy(acc.at[pl.ds(rows0, NPT)], out_hbm.at[c, pl.ds(rows0, NPT)])


def _node_mlp_body(x_ref, p0_ref, p1_ref, wh1x_ref, wh1m_ref, wh1n_ref,
                   bh1_ref, wh2_ref, bh2_ref, out_ref):
    xb = x_ref[...]
    p0 = p0_ref[...]
    p1 = p1_ref[...]
    mh = p0[:, 0:H] + p1[:, 0:H]
    mvp = p0[:, H:H + 16] + p1[:, H:H + 16]
    nrm = jnp.sqrt(jnp.sum(mvp * mvp, axis=1, keepdims=True) + 1e-12)
    pre = (jnp.dot(xb, wh1x_ref[...], preferred_element_type=jnp.float32)
           + jnp.dot(mh, wh1m_ref[...], preferred_element_type=jnp.float32)
           + nrm * wh1n_ref[...] + bh1_ref[...])
    u = jnp.dot(_silu(pre), wh2_ref[...], preferred_element_type=jnp.float32)
    out_ref[...] = xb + u + bh2_ref[...]


def _node_mlp(x, p0, p1, wh1x, wh1m, wh1n, bh1, wh2, bh2):
    nb = 1000
    grid = N // nb
    return pl.pallas_call(
        _node_mlp_body,
        grid=(grid,),
        in_specs=[
            pl.BlockSpec((nb, D), lambda i: (i, 0)),
            pl.BlockSpec((nb, MW), lambda i: (i, 0)),
            pl.BlockSpec((nb, MW), lambda i: (i, 0)),
            pl.BlockSpec((D, H), lambda i: (0, 0)),
            pl.BlockSpec((H, H), lambda i: (0, 0)),
            pl.BlockSpec((1, H), lambda i: (0, 0)),
            pl.BlockSpec((1, H), lambda i: (0, 0)),
            pl.BlockSpec((H, D), lambda i: (0, 0)),
            pl.BlockSpec((1, D), lambda i: (0, 0)),
        ],
        out_specs=pl.BlockSpec((nb, D), lambda i: (i, 0)),
        out_shape=jax.ShapeDtypeStruct((N, D), jnp.float32),
    )(x, p0, p1, wh1x, wh1m, wh1n, bh1, wh2, bh2)


def kernel(x, pos, vel, edge_index, params):
    we1, be1 = params['phi_e'][0]
    we2, be2 = params['phi_e'][1]
    we3, be3 = params['phi_e'][2]
    wv1, bv1 = params['phi_v'][0]
    wv2, bv2 = params['phi_v'][1]
    wh1, bh1 = params['phi_h'][0]
    wh2, bh2 = params['phi_h'][1]

    wsrc = jnp.concatenate([we1[D:2 * D], wv1[D:2 * D]], axis=1)
    wdst = jnp.concatenate([we1[0:D], wv1[0:D]], axis=1)
    b1 = jnp.concatenate([be1, bv1])[None, :]
    r512 = jnp.concatenate([we1[2 * D], wv1[2 * D]])[None, :]
    r513 = jnp.concatenate([we1[2 * D + 1], wv1[2 * D + 1]])[None, :]
    geom = jnp.concatenate(
        [pos, vel, jnp.zeros((N, 12), jnp.float32)], axis=1)

    src = edge_index[0]
    dst = edge_index[1]

    ts, td = _node_proj(x, wsrc, wdst, b1, geom)
    r = _edge_gather_kernel()(ts, td, src, dst)
    msg = _edge_mlp(r, we2, be2[None, :], we3, be3[None, :],
                    wv2, bv2[None, :], r512, r513)
    zeros = jnp.zeros((NP, MW), jnp.float32)
    partials = _edge_scatter_kernel()(msg, dst, zeros)

    out = _node_mlp(x, partials[0, :N], partials[1, :N],
                    wh1[0:D], wh1[D:D + H], wh1[D + H][None, :],
                    bh1[None, :], wh2, bh2[None, :])
    return out


# speedup vs baseline: 6.7156x; 1.4000x over previous
import functools

import jax
import jax.numpy as jnp
from jax import lax
from jax.experimental import pallas as pl
from jax.experimental.pallas import tpu as pltpu
from jax.experimental.pallas import tpu_sc as plsc

NC = 2
NS = 16
NW = NC * NS
LANES = 16

N = 10000
E = 160000
D = 256
H = 64

PW = 128
MW = 128
BG = 128
NBLK = E // BG
NP = 10240
NPT = NP // NS


def _silu(t):
    return t / (1.0 + jnp.exp(-t))


@functools.lru_cache(maxsize=None)
def _sc_mesh():
    return plsc.VectorSubcoreMesh(
        core_axis_name="c", subcore_axis_name="s",
        num_cores=NC, num_subcores=NS)


def _proj_body(x_ref, ws_ref, wd_ref, b1_ref, ts_ref, td_ref):
    xb = x_ref[...]
    ts_ref[...] = jnp.dot(xb, ws_ref[...], preferred_element_type=jnp.float32)
    td_ref[...] = (jnp.dot(xb, wd_ref[...], preferred_element_type=jnp.float32)
                   + b1_ref[...])


def _node_proj(x, wsrc, wdst, b1):
    nb = 1000
    grid = N // nb
    return pl.pallas_call(
        _proj_body,
        grid=(grid,),
        in_specs=[
            pl.BlockSpec((nb, D), lambda i: (i, 0)),
            pl.BlockSpec((D, PW), lambda i: (0, 0)),
            pl.BlockSpec((D, PW), lambda i: (0, 0)),
            pl.BlockSpec((1, PW), lambda i: (0, 0)),
        ],
        out_specs=[
            pl.BlockSpec((nb, PW), lambda i: (i, 0)),
            pl.BlockSpec((nb, PW), lambda i: (i, 0)),
        ],
        out_shape=[
            jax.ShapeDtypeStruct((N, PW), jnp.float32),
            jax.ShapeDtypeStruct((N, PW), jnp.float32),
        ],
    )(x, wsrc, wdst, b1)


@functools.lru_cache(maxsize=None)
def _edge_gather_kernel():
    return pl.kernel(
        _edge_gather_body,
        out_type=[
            jax.ShapeDtypeStruct((E, PW), jnp.float32),
            jax.ShapeDtypeStruct((8, E), jnp.float32),
        ],
        mesh=_sc_mesh(),
        compiler_params=pltpu.CompilerParams(needs_layout_passes=False),
        scratch_types=[
            pltpu.VMEM((4 * N,), jnp.float32),
            pltpu.VMEM((BG,), jnp.int32),
            pltpu.VMEM((BG,), jnp.int32),
            pltpu.VMEM((BG, PW), jnp.float32),
            pltpu.VMEM((BG, PW), jnp.float32),
            pltpu.VMEM((8, BG), jnp.float32),
            pltpu.SemaphoreType.DMA,
            pltpu.SemaphoreType.DMA,
        ],
    )


def _edge_gather_body(ts_hbm, td_hbm, src_hbm, dst_hbm, geom_hbm,
                      out_r, out_g, geomv, sidx, didx, bufa, bufb, gbuf,
                      sem1, sem2):
    c = lax.axis_index("c")
    s = lax.axis_index("s")
    wid = s * NC + c
    cnt = 39 + jnp.where(wid < 2, 1, 0)

    pltpu.sync_copy(geom_hbm, geomv)
    zero16 = jnp.zeros((LANES,), jnp.float32)
    for r2 in range(2, 8):
        for k2 in range(BG // LANES):
            gbuf[r2, pl.ds(k2 * LANES, LANES)] = zero16

    def blk(j, carry):
        ebase = pl.multiple_of((wid + 32 * j) * BG, 128)
        pltpu.sync_copy(src_hbm.at[pl.ds(ebase, BG)], sidx)
        pltpu.sync_copy(dst_hbm.at[pl.ds(ebase, BG)], didx)
        cp1 = pltpu.async_copy(ts_hbm.at[sidx], bufa, sem1)
        cp2 = pltpu.async_copy(td_hbm.at[didx], bufb, sem2)
        cp1.wait()
        cp2.wait()

        def row(i, carry2):
            for k2 in range(PW // LANES):
                sl = (i, pl.ds(k2 * LANES, LANES))
                bufa[sl] = bufa[sl] + bufb[sl]
            return carry2

        lax.fori_loop(0, BG, row, 0)

        for g in range(BG // LANES):
            gsl = pl.ds(g * LANES, LANES)
            a_s = sidx[gsl] * 4
            a_d = didx[gsl] * 4
            pxs = plsc.load_gather(geomv, [a_s])
            pys = plsc.load_gather(geomv, [a_s + 1])
            vxs = plsc.load_gather(geomv, [a_s + 2])
            vys = plsc.load_gather(geomv, [a_s + 3])
            pxd = plsc.load_gather(geomv, [a_d])
            pyd = plsc.load_gather(geomv, [a_d + 1])
            vxd = plsc.load_gather(geomv, [a_d + 2])
            vyd = plsc.load_gather(geomv, [a_d + 3])
            relx = pxs - pxd
            rely = pys - pyd
            rvx = vxs - vxd
            rvy = vys - vyd
            gbuf[0, gsl] = relx * relx + rely * rely
            gbuf[1, gsl] = rvx * relx + rvy * rely

        pltpu.sync_copy(bufa, out_r.at[pl.ds(ebase, BG)])
        pltpu.sync_copy(gbuf, out_g.at[:, pl.ds(ebase, BG)])
        return carry

    lax.fori_loop(0, cnt, blk, 0)


def _edge_mlp_body(r_ref, g_ref, m8_ref, we2_ref, be2_ref, we3_ref, be3_ref,
                   wv2_ref, bv2_ref, msg_ref):
    rb = r_ref[...]
    t = rb + lax.dot_general(g_ref[...], m8_ref[...], (((0,), (0,)), ((), ())),
                             preferred_element_type=jnp.float32)
    e1 = _silu(t[:, 0:H])
    v1 = _silu(t[:, H:2 * H])
    h2 = _silu(jnp.dot(e1, we2_ref[...], preferred_element_type=jnp.float32)
               + be2_ref[...])
    mh = jnp.dot(h2, we3_ref[...], preferred_element_type=jnp.float32) + be3_ref[...]
    vw = jnp.dot(v1, wv2_ref[...], preferred_element_type=jnp.float32) + bv2_ref[...]
    padm = jnp.zeros((rb.shape[0], MW - H - 1), jnp.float32)
    msg_ref[...] = jnp.concatenate([mh, vw, padm], axis=1)


def _edge_mlp(r, g, m8, we2, be2, we3, be3, wv2, bv2):
    eb = 3200
    grid = E // eb
    return pl.pallas_call(
        _edge_mlp_body,
        grid=(grid,),
        in_specs=[
            pl.BlockSpec((eb, PW), lambda i: (i, 0)),
            pl.BlockSpec((8, eb), lambda i: (0, i)),
            pl.BlockSpec((8, PW), lambda i: (0, 0)),
            pl.BlockSpec((H, H), lambda i: (0, 0)),
            pl.BlockSpec((1, H), lambda i: (0, 0)),
            pl.BlockSpec((H, H), lambda i: (0, 0)),
            pl.BlockSpec((1, H), lambda i: (0, 0)),
            pl.BlockSpec((H, 1), lambda i: (0, 0)),
            pl.BlockSpec((1, 1), lambda i: (0, 0)),
        ],
        out_specs=pl.BlockSpec((eb, MW), lambda i: (i, 0)),
        out_shape=jax.ShapeDtypeStruct((E, MW), jnp.float32),
    )(r, g, m8, we2, be2, we3, be3, wv2, bv2)


@functools.lru_cache(maxsize=None)
def _edge_scatter_kernel():
    return pl.kernel(
        _edge_scatter_body,
        out_type=jax.ShapeDtypeStruct((NC, NP, MW), jnp.float32),
        mesh=_sc_mesh(),
        compiler_params=pltpu.CompilerParams(needs_layout_passes=False),
        scratch_types=[
            pltpu.VMEM((2 * N,), jnp.float32),
            pltpu.VMEM((BG,), jnp.int32),
            pltpu.VMEM((BG,), jnp.int32),
            pltpu.VMEM((BG, MW), jnp.float32),
            pltpu.VMEM_SHARED((NP, MW), jnp.float32),
        ],
    )


def _edge_scatter_body(msg_hbm, src_hbm, dst_hbm, pos_hbm, zeros_hbm, out_hbm,
                       posv, sidx, didx, mbuf, acc):
    c = lax.axis_index("c")
    s = lax.axis_index("s")
    wid = s * NC + c
    cnt = 39 + jnp.where(wid < 2, 1, 0)
    rows0 = pl.multiple_of(s * NPT, 8)
    pltpu.sync_copy(pos_hbm, posv)
    pltpu.sync_copy(zeros_hbm.at[pl.ds(rows0, NPT)], acc.at[pl.ds(rows0, NPT)])
    plsc.subcore_barrier()

    rows_base = jnp.arange(LANES, dtype=jnp.int32)
    c64 = jnp.full((LANES,), 64, jnp.int32)
    c65 = jnp.full((LANES,), 65, jnp.int32)

    def blk(j, carry):
        ebase = pl.multiple_of((wid + 32 * j) * BG, 128)
        pltpu.sync_copy(src_hbm.at[pl.ds(ebase, BG)], sidx)
        pltpu.sync_copy(dst_hbm.at[pl.ds(ebase, BG)], didx)
        pltpu.sync_copy(msg_hbm.at[pl.ds(ebase, BG)], mbuf)

        for g in range(BG // LANES):
            gsl = pl.ds(g * LANES, LANES)
            rows = rows_base + g * LANES
            vw = plsc.load_gather(mbuf, [rows, c64])
            a_s = sidx[gsl] * 2
            a_d = didx[gsl] * 2
            pxs = plsc.load_gather(posv, [a_s])
            pys = plsc.load_gather(posv, [a_s + 1])
            pxd = plsc.load_gather(posv, [a_d])
            pyd = plsc.load_gather(posv, [a_d + 1])
            plsc.store_scatter(mbuf, [rows, c64], vw * (pxs - pxd))
            plsc.store_scatter(mbuf, [rows, c65], vw * (pys - pyd))

        pltpu.sync_copy(mbuf, acc.at[didx], add=True)
        return carry

    lax.fori_loop(0, cnt, blk, 0)
    plsc.subcore_barrier()
    pltpu.sync_copy(acc.at[pl.ds(rows0, NPT)], out_hbm.at[c, pl.ds(rows0, NPT)])


def _node_mlp_body(x_ref, p0_ref, p1_ref, wh1x_ref, wh1m_ref, wh1n_ref,
                   bh1_ref, wh2_ref, bh2_ref, out_ref):
    xb = x_ref[...]
    p0 = p0_ref[...]
    p1 = p1_ref[...]
    mh = p0[:, 0:H] + p1[:, 0:H]
    mvp = p0[:, H:H + 16] + p1[:, H:H + 16]
    nrm = jnp.sqrt(jnp.sum(mvp * mvp, axis=1, keepdims=True) + 1e-12)
    pre = (jnp.dot(xb, wh1x_ref[...], preferred_element_type=jnp.float32)
           + jnp.dot(mh, wh1m_ref[...], preferred_element_type=jnp.float32)
           + nrm * wh1n_ref[...] + bh1_ref[...])
    u = jnp.dot(_silu(pre), wh2_ref[...], preferred_element_type=jnp.float32)
    out_ref[...] = xb + u + bh2_ref[...]


def _node_mlp(x, p0, p1, wh1x, wh1m, wh1n, bh1, wh2, bh2):
    nb = 1000
    grid = N // nb
    return pl.pallas_call(
        _node_mlp_body,
        grid=(grid,),
        in_specs=[
            pl.BlockSpec((nb, D), lambda i: (i, 0)),
            pl.BlockSpec((nb, MW), lambda i: (i, 0)),
            pl.BlockSpec((nb, MW), lambda i: (i, 0)),
            pl.BlockSpec((D, H), lambda i: (0, 0)),
            pl.BlockSpec((H, H), lambda i: (0, 0)),
            pl.BlockSpec((1, H), lambda i: (0, 0)),
            pl.BlockSpec((1, H), lambda i: (0, 0)),
            pl.BlockSpec((H, D), lambda i: (0, 0)),
            pl.BlockSpec((1, D), lambda i: (0, 0)),
        ],
        out_specs=pl.BlockSpec((nb, D), lambda i: (i, 0)),
        out_shape=jax.ShapeDtypeStruct((N, D), jnp.float32),
    )(x, p0, p1, wh1x, wh1m, wh1n, bh1, wh2, bh2)


def kernel(x, pos, vel, edge_index, params):
    we1, be1 = params['phi_e'][0]
    we2, be2 = params['phi_e'][1]
    we3, be3 = params['phi_e'][2]
    wv1, bv1 = params['phi_v'][0]
    wv2, bv2 = params['phi_v'][1]
    wh1, bh1 = params['phi_h'][0]
    wh2, bh2 = params['phi_h'][1]

    wsrc = jnp.concatenate([we1[D:2 * D], wv1[D:2 * D]], axis=1)
    wdst = jnp.concatenate([we1[0:D], wv1[0:D]], axis=1)
    b1 = jnp.concatenate([be1, bv1])[None, :]
    r512 = jnp.concatenate([we1[2 * D], wv1[2 * D]])[None, :]
    r513 = jnp.concatenate([we1[2 * D + 1], wv1[2 * D + 1]])[None, :]
    m8 = jnp.concatenate(
        [r512, r513, jnp.zeros((6, PW), jnp.float32)], axis=0)
    geom4 = jnp.concatenate([pos, vel], axis=1).reshape(-1)
    pos2 = pos.reshape(-1)

    src = edge_index[0]
    dst = edge_index[1]

    ts, td = _node_proj(x, wsrc, wdst, b1)
    r, g = _edge_gather_kernel()(ts, td, src, dst, geom4)
    msg = _edge_mlp(r, g, m8, we2, be2[None, :], we3, be3[None, :],
                    wv2, bv2[None, :])
    zeros = jnp.zeros((NP, MW), jnp.float32)
    partials = _edge_scatter_kernel()(msg, src, dst, pos2, zeros)

    out = _node_mlp(x, partials[0, :N], partials[1, :N],
                    wh1[0:D], wh1[D:D + H], wh1[D + H][None, :],
                    bh1[None, :], wh2, bh2[None, :])
    return out


# speedup vs baseline: 8.3167x; 1.2384x over previous
import functools

import jax
import jax.numpy as jnp
from jax import lax
from jax.experimental import pallas as pl
from jax.experimental.pallas import tpu as pltpu
from jax.experimental.pallas import tpu_sc as plsc

NC = 2
NS = 16
NW = NC * NS
LANES = 16

N = 10000
E = 160000
D = 256
H = 64

PW = 128
MW = 128
BG = 128
NBLK = E // BG
NP = 10240
NPT = NP // NS


def _silu(t):
    return t / (1.0 + jnp.exp(-t))


@functools.lru_cache(maxsize=None)
def _sc_mesh():
    return plsc.VectorSubcoreMesh(
        core_axis_name="c", subcore_axis_name="s",
        num_cores=NC, num_subcores=NS)


def _proj_body(x_ref, ws_ref, wd_ref, b1_ref, ts_ref, td_ref):
    xb = x_ref[...]
    ts_ref[...] = jnp.dot(xb, ws_ref[...], preferred_element_type=jnp.float32)
    td_ref[...] = (jnp.dot(xb, wd_ref[...], preferred_element_type=jnp.float32)
                   + b1_ref[...])


def _node_proj(x, wsrc, wdst, b1):
    nb = 1000
    grid = N // nb
    return pl.pallas_call(
        _proj_body,
        grid=(grid,),
        in_specs=[
            pl.BlockSpec((nb, D), lambda i: (i, 0)),
            pl.BlockSpec((D, PW), lambda i: (0, 0)),
            pl.BlockSpec((D, PW), lambda i: (0, 0)),
            pl.BlockSpec((1, PW), lambda i: (0, 0)),
        ],
        out_specs=[
            pl.BlockSpec((nb, PW), lambda i: (i, 0)),
            pl.BlockSpec((nb, PW), lambda i: (i, 0)),
        ],
        out_shape=[
            jax.ShapeDtypeStruct((N, PW), jnp.float32),
            jax.ShapeDtypeStruct((N, PW), jnp.float32),
        ],
    )(x, wsrc, wdst, b1)


MAXB = 40


@functools.lru_cache(maxsize=None)
def _edge_gather_kernel():
    return pl.kernel(
        _edge_gather_body,
        out_type=[
            jax.ShapeDtypeStruct((E, PW), jnp.float32),
            jax.ShapeDtypeStruct((8, E), jnp.float32),
        ],
        mesh=_sc_mesh(),
        compiler_params=pltpu.CompilerParams(needs_layout_passes=False),
        scratch_types=[
            pltpu.VMEM((4 * N,), jnp.float32),
            pltpu.VMEM((MAXB * BG,), jnp.int32),
            pltpu.VMEM((MAXB * BG,), jnp.int32),
            [pltpu.VMEM((BG, PW), jnp.float32)] * 2,
            [pltpu.VMEM((BG, PW), jnp.float32)] * 2,
            [pltpu.VMEM((8, BG), jnp.float32)] * 2,
            [pltpu.SemaphoreType.DMA] * 2,
            [pltpu.SemaphoreType.DMA] * 2,
            pltpu.SemaphoreType.DMA,
        ],
    )


def _edge_gather_body(ts_hbm, td_hbm, src_hbm, dst_hbm, geom_hbm,
                      out_r, out_g, geomv, sidx_all, didx_all,
                      bufa, bufb, gbuf, gsem, wsem, isem):
    c = lax.axis_index("c")
    s = lax.axis_index("s")
    wid = s * NC + c
    cnt = 39 + jnp.where(wid < 2, 1, 0)

    cps = []
    for k in range(MAXB - 1):
        eb = pl.multiple_of((wid + 32 * k) * BG, 128)
        cps.append(pltpu.async_copy(
            src_hbm.at[pl.ds(eb, BG)], sidx_all.at[pl.ds(k * BG, BG)], isem))
        cps.append(pltpu.async_copy(
            dst_hbm.at[pl.ds(eb, BG)], didx_all.at[pl.ds(k * BG, BG)], isem))
    pltpu.sync_copy(geom_hbm, geomv)
    for cp in cps:
        cp.wait()

    @pl.when(wid < 2)
    def _():
        k = MAXB - 1
        eb = pl.multiple_of((wid + 32 * k) * BG, 128)
        pltpu.sync_copy(src_hbm.at[pl.ds(eb, BG)], sidx_all.at[pl.ds(k * BG, BG)])
        pltpu.sync_copy(dst_hbm.at[pl.ds(eb, BG)], didx_all.at[pl.ds(k * BG, BG)])

    zero16 = jnp.zeros((LANES,), jnp.float32)
    for b in range(2):
        for r2 in range(2, 8):
            for k2 in range(BG // LANES):
                gbuf[b][r2, pl.ds(k2 * LANES, LANES)] = zero16

    def fire(t, j):
        c1 = pltpu.async_copy(
            ts_hbm.at[sidx_all.at[pl.ds(j * BG, BG)]], bufa[t], gsem[t])
        c2 = pltpu.async_copy(
            td_hbm.at[didx_all.at[pl.ds(j * BG, BG)]], bufb[t], gsem[t])
        return c1, c2

    def wait_writes(t):
        pltpu.make_async_copy(bufa[t], out_r.at[pl.ds(0, BG)], wsem[t]).wait()
        pltpu.make_async_copy(gbuf[t], out_g.at[:, pl.ds(0, BG)], wsem[t]).wait()

    def process(t, j):
        pltpu.make_async_copy(
            ts_hbm.at[sidx_all.at[pl.ds(0, BG)]], bufa[t], gsem[t]).wait()
        pltpu.make_async_copy(
            td_hbm.at[didx_all.at[pl.ds(0, BG)]], bufb[t], gsem[t]).wait()

        def row(i, carry2):
            for k2 in range(PW // LANES):
                sl = (i, pl.ds(k2 * LANES, LANES))
                bufa[t][sl] = bufa[t][sl] + bufb[t][sl]
            return carry2

        lax.fori_loop(0, BG, row, 0)

        for g in range(BG // LANES):
            gsl = pl.ds(g * LANES, LANES)
            a_s = sidx_all[pl.ds(j * BG + g * LANES, LANES)] * 4
            a_d = didx_all[pl.ds(j * BG + g * LANES, LANES)] * 4
            pxs = plsc.load_gather(geomv, [a_s])
            pys = plsc.load_gather(geomv, [a_s + 1])
            vxs = plsc.load_gather(geomv, [a_s + 2])
            vys = plsc.load_gather(geomv, [a_s + 3])
            pxd = plsc.load_gather(geomv, [a_d])
            pyd = plsc.load_gather(geomv, [a_d + 1])
            vxd = plsc.load_gather(geomv, [a_d + 2])
            vyd = plsc.load_gather(geomv, [a_d + 3])
            relx = pxs - pxd
            rely = pys - pyd
            rvx = vxs - vxd
            rvy = vys - vyd
            gbuf[t][0, gsl] = relx * relx + rely * rely
            gbuf[t][1, gsl] = rvx * relx + rvy * rely

        ebase = pl.multiple_of((wid + 32 * j) * BG, 128)
        pltpu.async_copy(bufa[t], out_r.at[pl.ds(ebase, BG)], wsem[t])
        pltpu.async_copy(gbuf[t], out_g.at[:, pl.ds(ebase, BG)], wsem[t])

    fire(0, 0)

    def pair(p, carry):
        j0 = 2 * p
        @pl.when(p > 0)
        def _():
            wait_writes(1)
        fire(1, j0 + 1)
        process(0, j0)
        @pl.when(j0 + 2 < cnt)
        def _():
            wait_writes(0)
            fire(0, j0 + 2)
        process(1, j0 + 1)
        return carry

    lax.fori_loop(0, cnt // 2, pair, 0)

    @pl.when(cnt % 2 == 1)
    def _():
        process(0, cnt - 1)

    wait_writes(0)
    wait_writes(1)


def _edge_mlp_body(r_ref, g_ref, m8_ref, we2_ref, be2_ref, we3_ref, be3_ref,
                   wv2_ref, bv2_ref, msg_ref):
    rb = r_ref[...]
    t = rb + lax.dot_general(g_ref[...], m8_ref[...], (((0,), (0,)), ((), ())),
                             preferred_element_type=jnp.float32)
    e1 = _silu(t[:, 0:H])
    v1 = _silu(t[:, H:2 * H])
    h2 = _silu(jnp.dot(e1, we2_ref[...], preferred_element_type=jnp.float32)
               + be2_ref[...])
    mh = jnp.dot(h2, we3_ref[...], preferred_element_type=jnp.float32) + be3_ref[...]
    vw = jnp.dot(v1, wv2_ref[...], preferred_element_type=jnp.float32) + bv2_ref[...]
    padm = jnp.zeros((rb.shape[0], MW - H - 1), jnp.float32)
    msg_ref[...] = jnp.concatenate([mh, vw, padm], axis=1)


def _edge_mlp(r, g, m8, we2, be2, we3, be3, wv2, bv2):
    eb = 3200
    grid = E // eb
    return pl.pallas_call(
        _edge_mlp_body,
        grid=(grid,),
        in_specs=[
            pl.BlockSpec((eb, PW), lambda i: (i, 0)),
            pl.BlockSpec((8, eb), lambda i: (0, i)),
            pl.BlockSpec((8, PW), lambda i: (0, 0)),
            pl.BlockSpec((H, H), lambda i: (0, 0)),
            pl.BlockSpec((1, H), lambda i: (0, 0)),
            pl.BlockSpec((H, H), lambda i: (0, 0)),
            pl.BlockSpec((1, H), lambda i: (0, 0)),
            pl.BlockSpec((H, 1), lambda i: (0, 0)),
            pl.BlockSpec((1, 1), lambda i: (0, 0)),
        ],
        out_specs=pl.BlockSpec((eb, MW), lambda i: (i, 0)),
        out_shape=jax.ShapeDtypeStruct((E, MW), jnp.float32),
    )(r, g, m8, we2, be2, we3, be3, wv2, bv2)


@functools.lru_cache(maxsize=None)
def _edge_scatter_kernel():
    return pl.kernel(
        _edge_scatter_body,
        out_type=jax.ShapeDtypeStruct((NC, NP, MW), jnp.float32),
        mesh=_sc_mesh(),
        compiler_params=pltpu.CompilerParams(needs_layout_passes=False),
        scratch_types=[
            pltpu.VMEM((2 * N,), jnp.float32),
            pltpu.VMEM((BG,), jnp.int32),
            pltpu.VMEM((BG,), jnp.int32),
            pltpu.VMEM((BG, MW), jnp.float32),
            pltpu.VMEM_SHARED((NP, MW), jnp.float32),
        ],
    )


def _edge_scatter_body(msg_hbm, src_hbm, dst_hbm, pos_hbm, zeros_hbm, out_hbm,
                       posv, sidx, didx, mbuf, acc):
    c = lax.axis_index("c")
    s = lax.axis_index("s")
    wid = s * NC + c
    cnt = 39 + jnp.where(wid < 2, 1, 0)
    rows0 = pl.multiple_of(s * NPT, 8)
    pltpu.sync_copy(pos_hbm, posv)
    pltpu.sync_copy(zeros_hbm.at[pl.ds(rows0, NPT)], acc.at[pl.ds(rows0, NPT)])
    plsc.subcore_barrier()

    rows_base = jnp.arange(LANES, dtype=jnp.int32)
    c64 = jnp.full((LANES,), 64, jnp.int32)
    c65 = jnp.full((LANES,), 65, jnp.int32)

    def blk(j, carry):
        ebase = pl.multiple_of((wid + 32 * j) * BG, 128)
        pltpu.sync_copy(src_hbm.at[pl.ds(ebase, BG)], sidx)
        pltpu.sync_copy(dst_hbm.at[pl.ds(ebase, BG)], didx)
        pltpu.sync_copy(msg_hbm.at[pl.ds(ebase, BG)], mbuf)

        for g in range(BG // LANES):
            gsl = pl.ds(g * LANES, LANES)
            rows = rows_base + g * LANES
            vw = plsc.load_gather(mbuf, [rows, c64])
            a_s = sidx[gsl] * 2
            a_d = didx[gsl] * 2
            pxs = plsc.load_gather(posv, [a_s])
            pys = plsc.load_gather(posv, [a_s + 1])
            pxd = plsc.load_gather(posv, [a_d])
            pyd = plsc.load_gather(posv, [a_d + 1])
            plsc.store_scatter(mbuf, [rows, c64], vw * (pxs - pxd))
            plsc.store_scatter(mbuf, [rows, c65], vw * (pys - pyd))

        pltpu.sync_copy(mbuf, acc.at[didx], add=True)
        return carry

    lax.fori_loop(0, cnt, blk, 0)
    plsc.subcore_barrier()
    pltpu.sync_copy(acc.at[pl.ds(rows0, NPT)], out_hbm.at[c, pl.ds(rows0, NPT)])


def _node_mlp_body(x_ref, p0_ref, p1_ref, wh1x_ref, wh1m_ref, wh1n_ref,
                   bh1_ref, wh2_ref, bh2_ref, out_ref):
    xb = x_ref[...]
    p0 = p0_ref[...]
    p1 = p1_ref[...]
    mh = p0[:, 0:H] + p1[:, 0:H]
    mvp = p0[:, H:H + 16] + p1[:, H:H + 16]
    nrm = jnp.sqrt(jnp.sum(mvp * mvp, axis=1, keepdims=True) + 1e-12)
    pre = (jnp.dot(xb, wh1x_ref[...], preferred_element_type=jnp.float32)
           + jnp.dot(mh, wh1m_ref[...], preferred_element_type=jnp.float32)
           + nrm * wh1n_ref[...] + bh1_ref[...])
    u = jnp.dot(_silu(pre), wh2_ref[...], preferred_element_type=jnp.float32)
    out_ref[...] = xb + u + bh2_ref[...]


def _node_mlp(x, p0, p1, wh1x, wh1m, wh1n, bh1, wh2, bh2):
    nb = 1000
    grid = N // nb
    return pl.pallas_call(
        _node_mlp_body,
        grid=(grid,),
        in_specs=[
            pl.BlockSpec((nb, D), lambda i: (i, 0)),
            pl.BlockSpec((nb, MW), lambda i: (i, 0)),
            pl.BlockSpec((nb, MW), lambda i: (i, 0)),
            pl.BlockSpec((D, H), lambda i: (0, 0)),
            pl.BlockSpec((H, H), lambda i: (0, 0)),
            pl.BlockSpec((1, H), lambda i: (0, 0)),
            pl.BlockSpec((1, H), lambda i: (0, 0)),
            pl.BlockSpec((H, D), lambda i: (0, 0)),
            pl.BlockSpec((1, D), lambda i: (0, 0)),
        ],
        out_specs=pl.BlockSpec((nb, D), lambda i: (i, 0)),
        out_shape=jax.ShapeDtypeStruct((N, D), jnp.float32),
    )(x, p0, p1, wh1x, wh1m, wh1n, bh1, wh2, bh2)


def kernel(x, pos, vel, edge_index, params):
    we1, be1 = params['phi_e'][0]
    we2, be2 = params['phi_e'][1]
    we3, be3 = params['phi_e'][2]
    wv1, bv1 = params['phi_v'][0]
    wv2, bv2 = params['phi_v'][1]
    wh1, bh1 = params['phi_h'][0]
    wh2, bh2 = params['phi_h'][1]

    wsrc = jnp.concatenate([we1[D:2 * D], wv1[D:2 * D]], axis=1)
    wdst = jnp.concatenate([we1[0:D], wv1[0:D]], axis=1)
    b1 = jnp.concatenate([be1, bv1])[None, :]
    r512 = jnp.concatenate([we1[2 * D], wv1[2 * D]])[None, :]
    r513 = jnp.concatenate([we1[2 * D + 1], wv1[2 * D + 1]])[None, :]
    m8 = jnp.concatenate(
        [r512, r513, jnp.zeros((6, PW), jnp.float32)], axis=0)
    geom4 = jnp.concatenate([pos, vel], axis=1).reshape(-1)
    pos2 = pos.reshape(-1)

    src = edge_index[0]
    dst = edge_index[1]

    ts, td = _node_proj(x, wsrc, wdst, b1)
    r, g = _edge_gather_kernel()(ts, td, src, dst, geom4)
    msg = _edge_mlp(r, g, m8, we2, be2[None, :], we3, be3[None, :],
                    wv2, bv2[None, :])
    zeros = jnp.zeros((NP, MW), jnp.float32)
    partials = _edge_scatter_kernel()(msg, src, dst, pos2, zeros)

    out = _node_mlp(x, partials[0, :N], partials[1, :N],
                    wh1[0:D], wh1[D:D + H], wh1[D + H][None, :],
                    bh1[None, :], wh2, bh2[None, :])
    return out


# speedup vs baseline: 10.1931x; 1.2256x over previous
import functools

import jax
import jax.numpy as jnp
from jax import lax
from jax.experimental import pallas as pl
from jax.experimental.pallas import tpu as pltpu
from jax.experimental.pallas import tpu_sc as plsc

NC = 2
NS = 16
NW = NC * NS
LANES = 16

N = 10000
E = 160000
D = 256
H = 64

PW = 128
MW = 128
BG = 128
NBLK = E // BG
NP = 10240
NPT = NP // NS


def _silu(t):
    return t / (1.0 + jnp.exp(-t))


@functools.lru_cache(maxsize=None)
def _sc_mesh():
    return plsc.VectorSubcoreMesh(
        core_axis_name="c", subcore_axis_name="s",
        num_cores=NC, num_subcores=NS)


def _proj_body(x_ref, ws_ref, wd_ref, b1_ref, ts_ref, td_ref):
    xb = x_ref[...]
    ts_ref[...] = jnp.dot(xb, ws_ref[...], preferred_element_type=jnp.float32)
    td_ref[...] = (jnp.dot(xb, wd_ref[...], preferred_element_type=jnp.float32)
                   + b1_ref[...])


def _node_proj(x, wsrc, wdst, b1):
    nb = 1000
    grid = N // nb
    return pl.pallas_call(
        _proj_body,
        grid=(grid,),
        in_specs=[
            pl.BlockSpec((nb, D), lambda i: (i, 0)),
            pl.BlockSpec((D, PW), lambda i: (0, 0)),
            pl.BlockSpec((D, PW), lambda i: (0, 0)),
            pl.BlockSpec((1, PW), lambda i: (0, 0)),
        ],
        out_specs=[
            pl.BlockSpec((nb, PW), lambda i: (i, 0)),
            pl.BlockSpec((nb, PW), lambda i: (i, 0)),
        ],
        out_shape=[
            jax.ShapeDtypeStruct((N, PW), jnp.float32),
            jax.ShapeDtypeStruct((N, PW), jnp.float32),
        ],
    )(x, wsrc, wdst, b1)


MAXB = 40


@functools.lru_cache(maxsize=None)
def _edge_gather_kernel():
    return pl.kernel(
        _edge_gather_body,
        out_type=[
            jax.ShapeDtypeStruct((E, PW), jnp.float32),
            jax.ShapeDtypeStruct((8, E), jnp.float32),
        ],
        mesh=_sc_mesh(),
        compiler_params=pltpu.CompilerParams(needs_layout_passes=False),
        scratch_types=[
            pltpu.VMEM((4 * N,), jnp.float32),
            pltpu.VMEM((MAXB * BG,), jnp.int32),
            pltpu.VMEM((MAXB * BG,), jnp.int32),
            [pltpu.VMEM((BG, PW), jnp.float32)] * 2,
            [pltpu.VMEM((BG, PW), jnp.float32)] * 2,
            [pltpu.VMEM((8, BG), jnp.float32)] * 2,
            [pltpu.SemaphoreType.DMA] * 2,
            [pltpu.SemaphoreType.DMA] * 2,
            pltpu.SemaphoreType.DMA,
        ],
    )


def _edge_gather_body(ts_hbm, td_hbm, src_hbm, dst_hbm, geom_hbm,
                      out_r, out_g, geomv, sidx_all, didx_all,
                      bufa, bufb, gbuf, gsem, wsem, isem):
    c = lax.axis_index("c")
    s = lax.axis_index("s")
    wid = s * NC + c
    cnt = 39 + jnp.where(wid < 2, 1, 0)

    cps = []
    for k in range(MAXB - 1):
        eb = pl.multiple_of((wid + 32 * k) * BG, 128)
        cps.append(pltpu.async_copy(
            src_hbm.at[pl.ds(eb, BG)], sidx_all.at[pl.ds(k * BG, BG)], isem))
        cps.append(pltpu.async_copy(
            dst_hbm.at[pl.ds(eb, BG)], didx_all.at[pl.ds(k * BG, BG)], isem))
    pltpu.sync_copy(geom_hbm, geomv)
    for cp in cps:
        cp.wait()

    @pl.when(wid < 2)
    def _():
        k = MAXB - 1
        eb = pl.multiple_of((wid + 32 * k) * BG, 128)
        pltpu.sync_copy(src_hbm.at[pl.ds(eb, BG)], sidx_all.at[pl.ds(k * BG, BG)])
        pltpu.sync_copy(dst_hbm.at[pl.ds(eb, BG)], didx_all.at[pl.ds(k * BG, BG)])

    zero16 = jnp.zeros((LANES,), jnp.float32)
    for b in range(2):
        for r2 in range(4, 8):
            for k2 in range(BG // LANES):
                gbuf[b][r2, pl.ds(k2 * LANES, LANES)] = zero16

    def fire(t, j):
        c1 = pltpu.async_copy(
            ts_hbm.at[sidx_all.at[pl.ds(j * BG, BG)]], bufa[t], gsem[t])
        c2 = pltpu.async_copy(
            td_hbm.at[didx_all.at[pl.ds(j * BG, BG)]], bufb[t], gsem[t])
        return c1, c2

    def wait_writes(t):
        pltpu.make_async_copy(bufa[t], out_r.at[pl.ds(0, BG)], wsem[t]).wait()
        pltpu.make_async_copy(gbuf[t], out_g.at[:, pl.ds(0, BG)], wsem[t]).wait()

    def process(t, j):
        pltpu.make_async_copy(
            ts_hbm.at[sidx_all.at[pl.ds(0, BG)]], bufa[t], gsem[t]).wait()
        pltpu.make_async_copy(
            td_hbm.at[didx_all.at[pl.ds(0, BG)]], bufb[t], gsem[t]).wait()

        def row(i, carry2):
            for k2 in range(PW // LANES):
                sl = (i, pl.ds(k2 * LANES, LANES))
                bufa[t][sl] = bufa[t][sl] + bufb[t][sl]
            return carry2

        lax.fori_loop(0, BG, row, 0)

        for g in range(BG // LANES):
            gsl = pl.ds(g * LANES, LANES)
            a_s = sidx_all[pl.ds(j * BG + g * LANES, LANES)] * 4
            a_d = didx_all[pl.ds(j * BG + g * LANES, LANES)] * 4
            pxs = plsc.load_gather(geomv, [a_s])
            pys = plsc.load_gather(geomv, [a_s + 1])
            vxs = plsc.load_gather(geomv, [a_s + 2])
            vys = plsc.load_gather(geomv, [a_s + 3])
            pxd = plsc.load_gather(geomv, [a_d])
            pyd = plsc.load_gather(geomv, [a_d + 1])
            vxd = plsc.load_gather(geomv, [a_d + 2])
            vyd = plsc.load_gather(geomv, [a_d + 3])
            relx = pxs - pxd
            rely = pys - pyd
            rvx = vxs - vxd
            rvy = vys - vyd
            gbuf[t][0, gsl] = relx * relx + rely * rely
            gbuf[t][1, gsl] = rvx * relx + rvy * rely
            gbuf[t][2, gsl] = relx
            gbuf[t][3, gsl] = rely

        ebase = pl.multiple_of((wid + 32 * j) * BG, 128)
        pltpu.async_copy(bufa[t], out_r.at[pl.ds(ebase, BG)], wsem[t])
        pltpu.async_copy(gbuf[t], out_g.at[:, pl.ds(ebase, BG)], wsem[t])

    fire(0, 0)

    def pair(p, carry):
        j0 = 2 * p
        @pl.when(p > 0)
        def _():
            wait_writes(1)
        fire(1, j0 + 1)
        process(0, j0)
        @pl.when(j0 + 2 < cnt)
        def _():
            wait_writes(0)
            fire(0, j0 + 2)
        process(1, j0 + 1)
        return carry

    lax.fori_loop(0, cnt // 2, pair, 0)

    @pl.when(cnt % 2 == 1)
    def _():
        process(0, cnt - 1)

    wait_writes(0)
    wait_writes(1)


def _edge_mlp_body(r_ref, g_ref, m8_ref, we2_ref, be2_ref, we3_ref, be3_ref,
                   wv2_ref, bv2_ref, msg_ref):
    rb = r_ref[...]
    t = rb + lax.dot_general(g_ref[...], m8_ref[...], (((0,), (0,)), ((), ())),
                             preferred_element_type=jnp.float32)
    e1 = _silu(t[:, 0:H])
    v1 = _silu(t[:, H:2 * H])
    h2 = _silu(jnp.dot(e1, we2_ref[...], preferred_element_type=jnp.float32)
               + be2_ref[...])
    mh = jnp.dot(h2, we3_ref[...], preferred_element_type=jnp.float32) + be3_ref[...]
    vw = jnp.dot(v1, wv2_ref[...], preferred_element_type=jnp.float32) + bv2_ref[...]
    padm = jnp.zeros((rb.shape[0], MW - H - 1), jnp.float32)
    msg_ref[...] = jnp.concatenate([mh, vw, padm], axis=1)


def _edge_mlp(r, g, m8, we2, be2, we3, be3, wv2, bv2):
    eb = 3200
    grid = E // eb
    return pl.pallas_call(
        _edge_mlp_body,
        grid=(grid,),
        in_specs=[
            pl.BlockSpec((eb, PW), lambda i: (i, 0)),
            pl.BlockSpec((8, eb), lambda i: (0, i)),
            pl.BlockSpec((8, PW), lambda i: (0, 0)),
            pl.BlockSpec((H, H), lambda i: (0, 0)),
            pl.BlockSpec((1, H), lambda i: (0, 0)),
            pl.BlockSpec((H, H), lambda i: (0, 0)),
            pl.BlockSpec((1, H), lambda i: (0, 0)),
            pl.BlockSpec((H, 1), lambda i: (0, 0)),
            pl.BlockSpec((1, 1), lambda i: (0, 0)),
        ],
        out_specs=pl.BlockSpec((eb, MW), lambda i: (i, 0)),
        out_shape=jax.ShapeDtypeStruct((E, MW), jnp.float32),
    )(r, g, m8, we2, be2, we3, be3, wv2, bv2)


@functools.lru_cache(maxsize=None)
def _edge_scatter_kernel():
    return pl.kernel(
        _edge_scatter_body,
        out_type=jax.ShapeDtypeStruct((NC, NP, MW), jnp.float32),
        mesh=_sc_mesh(),
        compiler_params=pltpu.CompilerParams(needs_layout_passes=False),
        scratch_types=[
            [pltpu.VMEM((BG,), jnp.int32)] * 2,
            [pltpu.VMEM((BG, MW), jnp.float32)] * 2,
            [pltpu.VMEM((8, BG), jnp.float32)] * 2,
            pltpu.VMEM_SHARED((NP, MW), jnp.float32),
            [pltpu.SemaphoreType.DMA] * 2,
            [pltpu.SemaphoreType.DMA] * 2,
        ],
    )


def _edge_scatter_body(msg_hbm, dst_hbm, g_hbm, zeros_hbm, out_hbm,
                       didx_blk, mbuf, gblk, acc, msem, ssem):
    c = lax.axis_index("c")
    s = lax.axis_index("s")
    wid = s * NC + c
    cnt = 39 + jnp.where(wid < 2, 1, 0)
    rows0 = pl.multiple_of(s * NPT, 8)

    pltpu.sync_copy(zeros_hbm.at[pl.ds(rows0, NPT)], acc.at[pl.ds(rows0, NPT)])
    plsc.subcore_barrier()

    rows_base = jnp.arange(LANES, dtype=jnp.int32)
    c64 = jnp.full((LANES,), 64, jnp.int32)
    c65 = jnp.full((LANES,), 65, jnp.int32)

    def fire_load(t, j):
        ebase = pl.multiple_of((wid + 32 * j) * BG, 128)
        pltpu.async_copy(msg_hbm.at[pl.ds(ebase, BG)], mbuf[t], msem[t])
        pltpu.async_copy(dst_hbm.at[pl.ds(ebase, BG)], didx_blk[t], msem[t])
        pltpu.async_copy(g_hbm.at[:, pl.ds(ebase, BG)], gblk[t], msem[t])

    def wait_scat(t):
        pltpu.make_async_copy(mbuf[t], acc.at[didx_blk[t]], ssem[t]).wait()

    def process(t, j):
        pltpu.make_async_copy(
            msg_hbm.at[pl.ds(0, BG)], mbuf[t], msem[t]).wait()
        pltpu.make_async_copy(
            dst_hbm.at[pl.ds(0, BG)], didx_blk[t], msem[t]).wait()
        pltpu.make_async_copy(
            g_hbm.at[:, pl.ds(0, BG)], gblk[t], msem[t]).wait()

        for g in range(BG // LANES):
            gsl = pl.ds(g * LANES, LANES)
            rows = rows_base + g * LANES
            vw = plsc.load_gather(mbuf[t], [rows, c64])
            relx = gblk[t][2, gsl]
            rely = gblk[t][3, gsl]
            plsc.store_scatter(mbuf[t], [rows, c64], vw * relx)
            plsc.store_scatter(mbuf[t], [rows, c65], vw * rely)

        pltpu.async_copy(mbuf[t], acc.at[didx_blk[t]], ssem[t], add=True)

    fire_load(0, 0)

    def pair(p, carry):
        j0 = 2 * p
        @pl.when(p > 0)
        def _():
            wait_scat(1)
        fire_load(1, j0 + 1)
        process(0, j0)
        @pl.when(j0 + 2 < cnt)
        def _():
            wait_scat(0)
            fire_load(0, j0 + 2)
        process(1, j0 + 1)
        return carry

    lax.fori_loop(0, cnt // 2, pair, 0)

    @pl.when(cnt % 2 == 1)
    def _():
        process(0, cnt - 1)

    wait_scat(0)
    wait_scat(1)
    plsc.subcore_barrier()
    pltpu.sync_copy(acc.at[pl.ds(rows0, NPT)], out_hbm.at[c, pl.ds(rows0, NPT)])


def _node_mlp_body(x_ref, p0_ref, p1_ref, wh1x_ref, wh1m_ref, wh1n_ref,
                   bh1_ref, wh2_ref, bh2_ref, out_ref):
    xb = x_ref[...]
    p0 = p0_ref[...]
    p1 = p1_ref[...]
    mh = p0[:, 0:H] + p1[:, 0:H]
    mvp = p0[:, H:H + 16] + p1[:, H:H + 16]
    nrm = jnp.sqrt(jnp.sum(mvp * mvp, axis=1, keepdims=True) + 1e-12)
    pre = (jnp.dot(xb, wh1x_ref[...], preferred_element_type=jnp.float32)
           + jnp.dot(mh, wh1m_ref[...], preferred_element_type=jnp.float32)
           + nrm * wh1n_ref[...] + bh1_ref[...])
    u = jnp.dot(_silu(pre), wh2_ref[...], preferred_element_type=jnp.float32)
    out_ref[...] = xb + u + bh2_ref[...]


def _node_mlp(x, p0, p1, wh1x, wh1m, wh1n, bh1, wh2, bh2):
    nb = 1000
    grid = N // nb
    return pl.pallas_call(
        _node_mlp_body,
        grid=(grid,),
        in_specs=[
            pl.BlockSpec((nb, D), lambda i: (i, 0)),
            pl.BlockSpec((nb, MW), lambda i: (i, 0)),
            pl.BlockSpec((nb, MW), lambda i: (i, 0)),
            pl.BlockSpec((D, H), lambda i: (0, 0)),
            pl.BlockSpec((H, H), lambda i: (0, 0)),
            pl.BlockSpec((1, H), lambda i: (0, 0)),
            pl.BlockSpec((1, H), lambda i: (0, 0)),
            pl.BlockSpec((H, D), lambda i: (0, 0)),
            pl.BlockSpec((1, D), lambda i: (0, 0)),
        ],
        out_specs=pl.BlockSpec((nb, D), lambda i: (i, 0)),
        out_shape=jax.ShapeDtypeStruct((N, D), jnp.float32),
    )(x, p0, p1, wh1x, wh1m, wh1n, bh1, wh2, bh2)


def kernel(x, pos, vel, edge_index, params):
    we1, be1 = params['phi_e'][0]
    we2, be2 = params['phi_e'][1]
    we3, be3 = params['phi_e'][2]
    wv1, bv1 = params['phi_v'][0]
    wv2, bv2 = params['phi_v'][1]
    wh1, bh1 = params['phi_h'][0]
    wh2, bh2 = params['phi_h'][1]

    wsrc = jnp.concatenate([we1[D:2 * D], wv1[D:2 * D]], axis=1)
    wdst = jnp.concatenate([we1[0:D], wv1[0:D]], axis=1)
    b1 = jnp.concatenate([be1, bv1])[None, :]
    r512 = jnp.concatenate([we1[2 * D], wv1[2 * D]])[None, :]
    r513 = jnp.concatenate([we1[2 * D + 1], wv1[2 * D + 1]])[None, :]
    m8 = jnp.concatenate(
        [r512, r513, jnp.zeros((6, PW), jnp.float32)], axis=0)
    geom4 = jnp.concatenate([pos, vel], axis=1).reshape(-1)

    src = edge_index[0]
    dst = edge_index[1]

    ts, td = _node_proj(x, wsrc, wdst, b1)
    r, g = _edge_gather_kernel()(ts, td, src, dst, geom4)
    msg = _edge_mlp(r, g, m8, we2, be2[None, :], we3, be3[None, :],
                    wv2, bv2[None, :])
    zeros = jnp.zeros((NP, MW), jnp.float32)
    partials = _edge_scatter_kernel()(msg, dst, g, zeros)

    out = _node_mlp(x, partials[0, :N], partials[1, :N],
                    wh1[0:D], wh1[D:D + H], wh1[D + H][None, :],
                    bh1[None, :], wh2, bh2[None, :])
    return out


# speedup vs baseline: 10.4745x; 1.0276x over previous
import functools

import jax
import jax.numpy as jnp
from jax import lax
from jax.experimental import pallas as pl
from jax.experimental.pallas import tpu as pltpu
from jax.experimental.pallas import tpu_sc as plsc

NC = 2
NS = 16
NW = NC * NS
LANES = 16

N = 10000
E = 160000
D = 256
H = 64

PW = 128
MW = 128
BG = 128
NBLK = E // BG
NP = 10240
NPT = NP // NS


def _silu(t):
    return t / (1.0 + jnp.exp(-t))


@functools.lru_cache(maxsize=None)
def _sc_mesh():
    return plsc.VectorSubcoreMesh(
        core_axis_name="c", subcore_axis_name="s",
        num_cores=NC, num_subcores=NS)


def _proj_body(x_ref, ws_ref, wd_ref, b1_ref, ts_ref, td_ref):
    xb = x_ref[...]
    ts_ref[...] = jnp.dot(xb, ws_ref[...], preferred_element_type=jnp.float32)
    td_ref[...] = (jnp.dot(xb, wd_ref[...], preferred_element_type=jnp.float32)
                   + b1_ref[...])


def _node_proj(x, wsrc, wdst, b1):
    nb = 1000
    grid = N // nb
    return pl.pallas_call(
        _proj_body,
        grid=(grid,),
        in_specs=[
            pl.BlockSpec((nb, D), lambda i: (i, 0)),
            pl.BlockSpec((D, PW), lambda i: (0, 0)),
            pl.BlockSpec((D, PW), lambda i: (0, 0)),
            pl.BlockSpec((1, PW), lambda i: (0, 0)),
        ],
        out_specs=[
            pl.BlockSpec((nb, PW), lambda i: (i, 0)),
            pl.BlockSpec((nb, PW), lambda i: (i, 0)),
        ],
        out_shape=[
            jax.ShapeDtypeStruct((N, PW), jnp.float32),
            jax.ShapeDtypeStruct((N, PW), jnp.float32),
        ],
    )(x, wsrc, wdst, b1)


MAXB = 40


@functools.lru_cache(maxsize=None)
def _edge_gather_kernel():
    return pl.kernel(
        _edge_gather_body,
        out_type=[
            jax.ShapeDtypeStruct((E, PW), jnp.float32),
            jax.ShapeDtypeStruct((8, E), jnp.float32),
        ],
        mesh=_sc_mesh(),
        compiler_params=pltpu.CompilerParams(needs_layout_passes=False),
        scratch_types=[
            pltpu.VMEM((4 * N,), jnp.float32),
            pltpu.VMEM((MAXB * BG,), jnp.int32),
            pltpu.VMEM((MAXB * BG,), jnp.int32),
            [pltpu.VMEM((BG, PW), jnp.float32)] * 2,
            [pltpu.VMEM((BG, PW), jnp.float32)] * 2,
            [pltpu.VMEM((8, BG), jnp.float32)] * 2,
            [pltpu.SemaphoreType.DMA] * 2,
            [pltpu.SemaphoreType.DMA] * 2,
            pltpu.SemaphoreType.DMA,
        ],
    )


def _edge_gather_body(ts_hbm, td_hbm, src_hbm, dst_hbm, geom_hbm,
                      out_r, out_g, geomv, sidx_all, didx_all,
                      bufa, bufb, gbuf, gsem, wsem, isem):
    c = lax.axis_index("c")
    s = lax.axis_index("s")
    wid = s * NC + c
    cnt = 39 + jnp.where(wid < 2, 1, 0)

    cps = []
    for k in range(MAXB - 1):
        eb = pl.multiple_of((wid + 32 * k) * BG, 128)
        cps.append(pltpu.async_copy(
            src_hbm.at[pl.ds(eb, BG)], sidx_all.at[pl.ds(k * BG, BG)], isem))
        cps.append(pltpu.async_copy(
            dst_hbm.at[pl.ds(eb, BG)], didx_all.at[pl.ds(k * BG, BG)], isem))
    pltpu.sync_copy(geom_hbm, geomv)
    for cp in cps:
        cp.wait()

    @pl.when(wid < 2)
    def _():
        k = MAXB - 1
        eb = pl.multiple_of((wid + 32 * k) * BG, 128)
        pltpu.sync_copy(src_hbm.at[pl.ds(eb, BG)], sidx_all.at[pl.ds(k * BG, BG)])
        pltpu.sync_copy(dst_hbm.at[pl.ds(eb, BG)], didx_all.at[pl.ds(k * BG, BG)])

    zero16 = jnp.zeros((LANES,), jnp.float32)
    for b in range(2):
        for r2 in range(4, 8):
            for k2 in range(BG // LANES):
                gbuf[b][r2, pl.ds(k2 * LANES, LANES)] = zero16

    def fire(t, j):
        c1 = pltpu.async_copy(
            ts_hbm.at[sidx_all.at[pl.ds(j * BG, BG)]], bufa[t], gsem[t])
        c2 = pltpu.async_copy(
            td_hbm.at[didx_all.at[pl.ds(j * BG, BG)]], bufb[t], gsem[t])
        return c1, c2

    def wait_writes(t):
        pltpu.make_async_copy(bufa[t], out_r.at[pl.ds(0, BG)], wsem[t]).wait()
        pltpu.make_async_copy(gbuf[t], out_g.at[:, pl.ds(0, BG)], wsem[t]).wait()

    def process(t, j):
        pltpu.make_async_copy(
            ts_hbm.at[sidx_all.at[pl.ds(0, BG)]], bufa[t], gsem[t]).wait()
        pltpu.make_async_copy(
            td_hbm.at[didx_all.at[pl.ds(0, BG)]], bufb[t], gsem[t]).wait()

        def row(i, carry2):
            for k2 in range(PW // LANES):
                sl = (i, pl.ds(k2 * LANES, LANES))
                bufa[t][sl] = bufa[t][sl] + bufb[t][sl]
            return carry2

        lax.fori_loop(0, BG, row, 0)

        for g in range(BG // LANES):
            gsl = pl.ds(g * LANES, LANES)
            a_s = sidx_all[pl.ds(j * BG + g * LANES, LANES)] * 4
            a_d = didx_all[pl.ds(j * BG + g * LANES, LANES)] * 4
            pxs = plsc.load_gather(geomv, [a_s])
            pys = plsc.load_gather(geomv, [a_s + 1])
            vxs = plsc.load_gather(geomv, [a_s + 2])
            vys = plsc.load_gather(geomv, [a_s + 3])
            pxd = plsc.load_gather(geomv, [a_d])
            pyd = plsc.load_gather(geomv, [a_d + 1])
            vxd = plsc.load_gather(geomv, [a_d + 2])
            vyd = plsc.load_gather(geomv, [a_d + 3])
            relx = pxs - pxd
            rely = pys - pyd
            rvx = vxs - vxd
            rvy = vys - vyd
            gbuf[t][0, gsl] = relx * relx + rely * rely
            gbuf[t][1, gsl] = rvx * relx + rvy * rely
            gbuf[t][2, gsl] = relx
            gbuf[t][3, gsl] = rely

        ebase = pl.multiple_of((wid + 32 * j) * BG, 128)
        pltpu.async_copy(bufa[t], out_r.at[pl.ds(ebase, BG)], wsem[t])
        pltpu.async_copy(gbuf[t], out_g.at[:, pl.ds(ebase, BG)], wsem[t])

    fire(0, 0)

    def pair(p, carry):
        j0 = 2 * p
        @pl.when(p > 0)
        def _():
            wait_writes(1)
        fire(1, j0 + 1)
        process(0, j0)
        @pl.when(j0 + 2 < cnt)
        def _():
            wait_writes(0)
            fire(0, j0 + 2)
        process(1, j0 + 1)
        return carry

    lax.fori_loop(0, cnt // 2, pair, 0)

    @pl.when(cnt % 2 == 1)
    def _():
        process(0, cnt - 1)

    wait_writes(0)
    wait_writes(1)


def _edge_mlp_body(r_ref, g_ref, m8_ref, wc_ref, bc_ref, we3_ref, be3_ref,
                   msg_ref):
    rb = r_ref[...]
    t = rb + lax.dot_general(g_ref[...], m8_ref[...], (((0,), (0,)), ((), ())),
                             preferred_element_type=jnp.float32)
    a = _silu(t)
    t2 = jnp.dot(a, wc_ref[...], preferred_element_type=jnp.float32) + bc_ref[...]
    mh = (jnp.dot(_silu(t2[:, 0:H]), we3_ref[...],
                  preferred_element_type=jnp.float32) + be3_ref[...])
    msg_ref[...] = jnp.concatenate([mh, t2[:, H:MW]], axis=1)


def _edge_mlp(r, g, m8, wc, bc, we3, be3):
    eb = 3200
    grid = E // eb
    return pl.pallas_call(
        _edge_mlp_body,
        grid=(grid,),
        in_specs=[
            pl.BlockSpec((eb, PW), lambda i: (i, 0)),
            pl.BlockSpec((8, eb), lambda i: (0, i)),
            pl.BlockSpec((8, PW), lambda i: (0, 0)),
            pl.BlockSpec((PW, PW), lambda i: (0, 0)),
            pl.BlockSpec((1, PW), lambda i: (0, 0)),
            pl.BlockSpec((H, H), lambda i: (0, 0)),
            pl.BlockSpec((1, H), lambda i: (0, 0)),
        ],
        out_specs=pl.BlockSpec((eb, MW), lambda i: (i, 0)),
        out_shape=jax.ShapeDtypeStruct((E, MW), jnp.float32),
    )(r, g, m8, wc, bc, we3, be3)


@functools.lru_cache(maxsize=None)
def _edge_scatter_kernel():
    return pl.kernel(
        _edge_scatter_body,
        out_type=jax.ShapeDtypeStruct((NC, NP, MW), jnp.float32),
        mesh=_sc_mesh(),
        compiler_params=pltpu.CompilerParams(needs_layout_passes=False),
        scratch_types=[
            [pltpu.VMEM((BG,), jnp.int32)] * 2,
            [pltpu.VMEM((BG, MW), jnp.float32)] * 2,
            [pltpu.VMEM((8, BG), jnp.float32)] * 2,
            pltpu.VMEM_SHARED((NP, MW), jnp.float32),
            [pltpu.SemaphoreType.DMA] * 2,
            [pltpu.SemaphoreType.DMA] * 2,
        ],
    )


def _edge_scatter_body(msg_hbm, dst_hbm, g_hbm, zeros_hbm, out_hbm,
                       didx_blk, mbuf, gblk, acc, msem, ssem):
    c = lax.axis_index("c")
    s = lax.axis_index("s")
    wid = s * NC + c
    cnt = 39 + jnp.where(wid < 2, 1, 0)
    rows0 = pl.multiple_of(s * NPT, 8)

    pltpu.sync_copy(zeros_hbm.at[pl.ds(rows0, NPT)], acc.at[pl.ds(rows0, NPT)])
    plsc.subcore_barrier()

    rows_base = jnp.arange(LANES, dtype=jnp.int32)
    c64 = jnp.full((LANES,), 64, jnp.int32)
    c65 = jnp.full((LANES,), 65, jnp.int32)

    def fire_load(t, j):
        ebase = pl.multiple_of((wid + 32 * j) * BG, 128)
        pltpu.async_copy(msg_hbm.at[pl.ds(ebase, BG)], mbuf[t], msem[t])
        pltpu.async_copy(dst_hbm.at[pl.ds(ebase, BG)], didx_blk[t], msem[t])
        pltpu.async_copy(g_hbm.at[:, pl.ds(ebase, BG)], gblk[t], msem[t])

    def wait_scat(t):
        pltpu.make_async_copy(mbuf[t], acc.at[didx_blk[t]], ssem[t]).wait()

    def process(t, j):
        pltpu.make_async_copy(
            msg_hbm.at[pl.ds(0, BG)], mbuf[t], msem[t]).wait()
        pltpu.make_async_copy(
            dst_hbm.at[pl.ds(0, BG)], didx_blk[t], msem[t]).wait()
        pltpu.make_async_copy(
            g_hbm.at[:, pl.ds(0, BG)], gblk[t], msem[t]).wait()

        for g in range(BG // LANES):
            gsl = pl.ds(g * LANES, LANES)
            rows = rows_base + g * LANES
            vw = plsc.load_gather(mbuf[t], [rows, c64])
            relx = gblk[t][2, gsl]
            rely = gblk[t][3, gsl]
            plsc.store_scatter(mbuf[t], [rows, c64], vw * relx)
            plsc.store_scatter(mbuf[t], [rows, c65], vw * rely)

        pltpu.async_copy(mbuf[t], acc.at[didx_blk[t]], ssem[t], add=True)

    fire_load(0, 0)

    def pair(p, carry):
        j0 = 2 * p
        @pl.when(p > 0)
        def _():
            wait_scat(1)
        fire_load(1, j0 + 1)
        process(0, j0)
        @pl.when(j0 + 2 < cnt)
        def _():
            wait_scat(0)
            fire_load(0, j0 + 2)
        process(1, j0 + 1)
        return carry

    lax.fori_loop(0, cnt // 2, pair, 0)

    @pl.when(cnt % 2 == 1)
    def _():
        process(0, cnt - 1)

    wait_scat(0)
    wait_scat(1)
    plsc.subcore_barrier()
    pltpu.sync_copy(acc.at[pl.ds(rows0, NPT)], out_hbm.at[c, pl.ds(rows0, NPT)])


def _node_mlp_body(x_ref, p0_ref, p1_ref, wh1x_ref, wh1m_ref, wh1n_ref,
                   bh1_ref, wh2_ref, bh2_ref, out_ref):
    xb = x_ref[...]
    p0 = p0_ref[...]
    p1 = p1_ref[...]
    mh = p0[:, 0:H] + p1[:, 0:H]
    mvp = p0[:, H:H + 16] + p1[:, H:H + 16]
    nrm = jnp.sqrt(jnp.sum(mvp * mvp, axis=1, keepdims=True) + 1e-12)
    pre = (jnp.dot(xb, wh1x_ref[...], preferred_element_type=jnp.float32)
           + jnp.dot(mh, wh1m_ref[...], preferred_element_type=jnp.float32)
           + nrm * wh1n_ref[...] + bh1_ref[...])
    u = jnp.dot(_silu(pre), wh2_ref[...], preferred_element_type=jnp.float32)
    out_ref[...] = xb + u + bh2_ref[...]


def _node_mlp(x, p0, p1, wh1x, wh1m, wh1n, bh1, wh2, bh2):
    nb = 1000
    grid = N // nb
    return pl.pallas_call(
        _node_mlp_body,
        grid=(grid,),
        in_specs=[
            pl.BlockSpec((nb, D), lambda i: (i, 0)),
            pl.BlockSpec((nb, MW), lambda i: (i, 0)),
            pl.BlockSpec((nb, MW), lambda i: (i, 0)),
            pl.BlockSpec((D, H), lambda i: (0, 0)),
            pl.BlockSpec((H, H), lambda i: (0, 0)),
            pl.BlockSpec((1, H), lambda i: (0, 0)),
            pl.BlockSpec((1, H), lambda i: (0, 0)),
            pl.BlockSpec((H, D), lambda i: (0, 0)),
            pl.BlockSpec((1, D), lambda i: (0, 0)),
        ],
        out_specs=pl.BlockSpec((nb, D), lambda i: (i, 0)),
        out_shape=jax.ShapeDtypeStruct((N, D), jnp.float32),
    )(x, p0, p1, wh1x, wh1m, wh1n, bh1, wh2, bh2)


def kernel(x, pos, vel, edge_index, params):
    we1, be1 = params['phi_e'][0]
    we2, be2 = params['phi_e'][1]
    we3, be3 = params['phi_e'][2]
    wv1, bv1 = params['phi_v'][0]
    wv2, bv2 = params['phi_v'][1]
    wh1, bh1 = params['phi_h'][0]
    wh2, bh2 = params['phi_h'][1]

    wsrc = jnp.concatenate([we1[D:2 * D], wv1[D:2 * D]], axis=1)
    wdst = jnp.concatenate([we1[0:D], wv1[0:D]], axis=1)
    b1 = jnp.concatenate([be1, bv1])[None, :]
    r512 = jnp.concatenate([we1[2 * D], wv1[2 * D]])[None, :]
    r513 = jnp.concatenate([we1[2 * D + 1], wv1[2 * D + 1]])[None, :]
    m8 = jnp.concatenate(
        [r512, r513, jnp.zeros((6, PW), jnp.float32)], axis=0)
    wc = jnp.zeros((PW, PW), jnp.float32)
    wc = wc.at[0:H, 0:H].set(we2)
    wc = wc.at[H:2 * H, H:H + 1].set(wv2)
    bc = jnp.zeros((1, PW), jnp.float32)
    bc = bc.at[0, 0:H].set(be2)
    bc = bc.at[0, H].set(bv2[0])
    geom4 = jnp.concatenate([pos, vel], axis=1).reshape(-1)

    src = edge_index[0]
    dst = edge_index[1]

    ts, td = _node_proj(x, wsrc, wdst, b1)
    r, g = _edge_gather_kernel()(ts, td, src, dst, geom4)
    msg = _edge_mlp(r, g, m8, wc, bc, we3, be3[None, :])
    zeros = jnp.zeros((NP, MW), jnp.float32)
    partials = _edge_scatter_kernel()(msg, dst, g, zeros)

    out = _node_mlp(x, partials[0, :N], partials[1, :N],
                    wh1[0:D], wh1[D:D + H], wh1[D + H][None, :],
                    bh1[None, :], wh2, bh2[None, :])
    return out


# speedup vs baseline: 11.0322x; 1.0532x over previous
import functools

import jax
import jax.numpy as jnp
from jax import lax
from jax.experimental import pallas as pl
from jax.experimental.pallas import tpu as pltpu
from jax.experimental.pallas import tpu_sc as plsc

NC = 2
NS = 16
NW = NC * NS
LANES = 16

N = 10000
E = 160000
D = 256
H = 64

PW = 128
MW = 128
BG = 128
NBLK = E // BG
NP = 10240
NPT = NP // NS


def _silu(t):
    return t / (1.0 + jnp.exp(-t))


@functools.lru_cache(maxsize=None)
def _sc_mesh():
    return plsc.VectorSubcoreMesh(
        core_axis_name="c", subcore_axis_name="s",
        num_cores=NC, num_subcores=NS)


def _proj_body(x_ref, ws_ref, wd_ref, b1_ref, ts_ref, td_ref):
    xb = x_ref[...]
    ts_ref[...] = jnp.dot(xb, ws_ref[...], preferred_element_type=jnp.float32)
    td_ref[...] = (jnp.dot(xb, wd_ref[...], preferred_element_type=jnp.float32)
                   + b1_ref[...])


def _node_proj(x, wsrc, wdst, b1):
    nb = 2000
    grid = N // nb
    return pl.pallas_call(
        _proj_body,
        grid=(grid,),
        in_specs=[
            pl.BlockSpec((nb, D), lambda i: (i, 0)),
            pl.BlockSpec((D, PW), lambda i: (0, 0)),
            pl.BlockSpec((D, PW), lambda i: (0, 0)),
            pl.BlockSpec((1, PW), lambda i: (0, 0)),
        ],
        out_specs=[
            pl.BlockSpec((nb, PW), lambda i: (i, 0)),
            pl.BlockSpec((nb, PW), lambda i: (i, 0)),
        ],
        out_shape=[
            jax.ShapeDtypeStruct((N, PW), jnp.float32),
            jax.ShapeDtypeStruct((N, PW), jnp.float32),
        ],
    )(x, wsrc, wdst, b1)


MAXB = 40


@functools.lru_cache(maxsize=None)
def _edge_gather_kernel():
    return pl.kernel(
        _edge_gather_body,
        out_type=[
            jax.ShapeDtypeStruct((E, PW), jnp.float32),
            jax.ShapeDtypeStruct((8, E), jnp.float32),
        ],
        mesh=_sc_mesh(),
        compiler_params=pltpu.CompilerParams(needs_layout_passes=False),
        scratch_types=[
            pltpu.VMEM((4 * N,), jnp.float32),
            pltpu.VMEM((MAXB * BG,), jnp.int32),
            pltpu.VMEM((MAXB * BG,), jnp.int32),
            [pltpu.VMEM((BG, PW), jnp.float32)] * 2,
            [pltpu.VMEM((BG, PW), jnp.float32)] * 2,
            [pltpu.VMEM((8, BG), jnp.float32)] * 2,
            [pltpu.SemaphoreType.DMA] * 2,
            [pltpu.SemaphoreType.DMA] * 2,
            pltpu.SemaphoreType.DMA,
        ],
    )


def _edge_gather_body(ts_hbm, td_hbm, src_hbm, dst_hbm, geom_hbm,
                      out_r, out_g, geomv, sidx_all, didx_all,
                      bufa, bufb, gbuf, gsem, wsem, isem):
    c = lax.axis_index("c")
    s = lax.axis_index("s")
    wid = s * NC + c
    cnt = 39 + jnp.where(wid < 2, 1, 0)

    cps = []
    for k in range(MAXB - 1):
        eb = pl.multiple_of((wid + 32 * k) * BG, 128)
        cps.append(pltpu.async_copy(
            src_hbm.at[pl.ds(eb, BG)], sidx_all.at[pl.ds(k * BG, BG)], isem))
        cps.append(pltpu.async_copy(
            dst_hbm.at[pl.ds(eb, BG)], didx_all.at[pl.ds(k * BG, BG)], isem))
    pltpu.sync_copy(geom_hbm, geomv)
    for cp in cps:
        cp.wait()

    @pl.when(wid < 2)
    def _():
        k = MAXB - 1
        eb = pl.multiple_of((wid + 32 * k) * BG, 128)
        pltpu.sync_copy(src_hbm.at[pl.ds(eb, BG)], sidx_all.at[pl.ds(k * BG, BG)])
        pltpu.sync_copy(dst_hbm.at[pl.ds(eb, BG)], didx_all.at[pl.ds(k * BG, BG)])

    zero16 = jnp.zeros((LANES,), jnp.float32)
    for b in range(2):
        for r2 in range(4, 8):
            for k2 in range(BG // LANES):
                gbuf[b][r2, pl.ds(k2 * LANES, LANES)] = zero16

    def fire(t, j):
        c1 = pltpu.async_copy(
            ts_hbm.at[sidx_all.at[pl.ds(j * BG, BG)]], bufa[t], gsem[t])
        c2 = pltpu.async_copy(
            td_hbm.at[didx_all.at[pl.ds(j * BG, BG)]], bufb[t], gsem[t])
        return c1, c2

    def wait_writes(t):
        pltpu.make_async_copy(bufa[t], out_r.at[pl.ds(0, BG)], wsem[t]).wait()
        pltpu.make_async_copy(gbuf[t], out_g.at[:, pl.ds(0, BG)], wsem[t]).wait()

    def process(t, j):
        pltpu.make_async_copy(
            ts_hbm.at[sidx_all.at[pl.ds(0, BG)]], bufa[t], gsem[t]).wait()
        pltpu.make_async_copy(
            td_hbm.at[didx_all.at[pl.ds(0, BG)]], bufb[t], gsem[t]).wait()

        def row(i, carry2):
            for k2 in range(PW // LANES):
                sl = (i, pl.ds(k2 * LANES, LANES))
                bufa[t][sl] = bufa[t][sl] + bufb[t][sl]
            return carry2

        lax.fori_loop(0, BG, row, 0)

        for g in range(BG // LANES):
            gsl = pl.ds(g * LANES, LANES)
            a_s = sidx_all[pl.ds(j * BG + g * LANES, LANES)] * 4
            a_d = didx_all[pl.ds(j * BG + g * LANES, LANES)] * 4
            pxs = plsc.load_gather(geomv, [a_s])
            pys = plsc.load_gather(geomv, [a_s + 1])
            vxs = plsc.load_gather(geomv, [a_s + 2])
            vys = plsc.load_gather(geomv, [a_s + 3])
            pxd = plsc.load_gather(geomv, [a_d])
            pyd = plsc.load_gather(geomv, [a_d + 1])
            vxd = plsc.load_gather(geomv, [a_d + 2])
            vyd = plsc.load_gather(geomv, [a_d + 3])
            relx = pxs - pxd
            rely = pys - pyd
            rvx = vxs - vxd
            rvy = vys - vyd
            gbuf[t][0, gsl] = relx * relx + rely * rely
            gbuf[t][1, gsl] = rvx * relx + rvy * rely
            gbuf[t][2, gsl] = relx
            gbuf[t][3, gsl] = rely

        ebase = pl.multiple_of((wid + 32 * j) * BG, 128)
        pltpu.async_copy(bufa[t], out_r.at[pl.ds(ebase, BG)], wsem[t])
        pltpu.async_copy(gbuf[t], out_g.at[:, pl.ds(ebase, BG)], wsem[t])

    fire(0, 0)

    def pair(p, carry):
        j0 = 2 * p
        @pl.when(p > 0)
        def _():
            wait_writes(1)
        fire(1, j0 + 1)
        process(0, j0)
        @pl.when(j0 + 2 < cnt)
        def _():
            wait_writes(0)
            fire(0, j0 + 2)
        process(1, j0 + 1)
        return carry

    lax.fori_loop(0, cnt // 2, pair, 0)

    @pl.when(cnt % 2 == 1)
    def _():
        process(0, cnt - 1)

    wait_writes(0)
    wait_writes(1)


def _edge_mlp_body(r_ref, g_ref, m8_ref, wc_ref, bc_ref, we3_ref, be3_ref,
                   msg_ref):
    rb = r_ref[...]
    t = rb + lax.dot_general(g_ref[...], m8_ref[...], (((0,), (0,)), ((), ())),
                             preferred_element_type=jnp.float32)
    a = _silu(t)
    t2 = jnp.dot(a, wc_ref[...], preferred_element_type=jnp.float32) + bc_ref[...]
    mh = (jnp.dot(_silu(t2[:, 0:H]), we3_ref[...],
                  preferred_element_type=jnp.float32) + be3_ref[...])
    msg_ref[...] = jnp.concatenate([mh, t2[:, H:MW]], axis=1)


def _edge_mlp(r, g, m8, wc, bc, we3, be3):
    eb = 6400
    grid = E // eb
    return pl.pallas_call(
        _edge_mlp_body,
        grid=(grid,),
        in_specs=[
            pl.BlockSpec((eb, PW), lambda i: (i, 0)),
            pl.BlockSpec((8, eb), lambda i: (0, i)),
            pl.BlockSpec((8, PW), lambda i: (0, 0)),
            pl.BlockSpec((PW, PW), lambda i: (0, 0)),
            pl.BlockSpec((1, PW), lambda i: (0, 0)),
            pl.BlockSpec((H, H), lambda i: (0, 0)),
            pl.BlockSpec((1, H), lambda i: (0, 0)),
        ],
        out_specs=pl.BlockSpec((eb, MW), lambda i: (i, 0)),
        out_shape=jax.ShapeDtypeStruct((E, MW), jnp.float32),
    )(r, g, m8, wc, bc, we3, be3)


@functools.lru_cache(maxsize=None)
def _edge_scatter_kernel():
    return pl.kernel(
        _edge_scatter_body,
        out_type=jax.ShapeDtypeStruct((NC, NP, MW), jnp.float32),
        mesh=_sc_mesh(),
        compiler_params=pltpu.CompilerParams(needs_layout_passes=False),
        scratch_types=[
            [pltpu.VMEM((BG,), jnp.int32)] * 2,
            [pltpu.VMEM((BG, MW), jnp.float32)] * 2,
            [pltpu.VMEM((8, BG), jnp.float32)] * 2,
            pltpu.VMEM_SHARED((NP, MW), jnp.float32),
            [pltpu.SemaphoreType.DMA] * 2,
            [pltpu.SemaphoreType.DMA] * 2,
        ],
    )


def _edge_scatter_body(msg_hbm, dst_hbm, g_hbm, zeros_hbm, out_hbm,
                       didx_blk, mbuf, gblk, acc, msem, ssem):
    c = lax.axis_index("c")
    s = lax.axis_index("s")
    wid = s * NC + c
    cnt = 39 + jnp.where(wid < 2, 1, 0)
    rows0 = pl.multiple_of(s * NPT, 8)

    pltpu.sync_copy(zeros_hbm.at[pl.ds(rows0, NPT)], acc.at[pl.ds(rows0, NPT)])
    plsc.subcore_barrier()

    rows_base = jnp.arange(LANES, dtype=jnp.int32)
    c64 = jnp.full((LANES,), 64, jnp.int32)
    c65 = jnp.full((LANES,), 65, jnp.int32)

    def fire_load(t, j):
        ebase = pl.multiple_of((wid + 32 * j) * BG, 128)
        pltpu.async_copy(msg_hbm.at[pl.ds(ebase, BG)], mbuf[t], msem[t])
        pltpu.async_copy(dst_hbm.at[pl.ds(ebase, BG)], didx_blk[t], msem[t])
        pltpu.async_copy(g_hbm.at[:, pl.ds(ebase, BG)], gblk[t], msem[t])

    def wait_scat(t):
        pltpu.make_async_copy(mbuf[t], acc.at[didx_blk[t]], ssem[t]).wait()

    def process(t, j):
        pltpu.make_async_copy(
            msg_hbm.at[pl.ds(0, BG)], mbuf[t], msem[t]).wait()
        pltpu.make_async_copy(
            dst_hbm.at[pl.ds(0, BG)], didx_blk[t], msem[t]).wait()
        pltpu.make_async_copy(
            g_hbm.at[:, pl.ds(0, BG)], gblk[t], msem[t]).wait()

        for g in range(BG // LANES):
            gsl = pl.ds(g * LANES, LANES)
            rows = rows_base + g * LANES
            vw = plsc.load_gather(mbuf[t], [rows, c64])
            relx = gblk[t][2, gsl]
            rely = gblk[t][3, gsl]
            plsc.store_scatter(mbuf[t], [rows, c64], vw * relx)
            plsc.store_scatter(mbuf[t], [rows, c65], vw * rely)

        pltpu.async_copy(mbuf[t], acc.at[didx_blk[t]], ssem[t], add=True)

    fire_load(0, 0)

    def pair(p, carry):
        j0 = 2 * p
        @pl.when(p > 0)
        def _():
            wait_scat(1)
        fire_load(1, j0 + 1)
        process(0, j0)
        @pl.when(j0 + 2 < cnt)
        def _():
            wait_scat(0)
            fire_load(0, j0 + 2)
        process(1, j0 + 1)
        return carry

    lax.fori_loop(0, cnt // 2, pair, 0)

    @pl.when(cnt % 2 == 1)
    def _():
        process(0, cnt - 1)

    wait_scat(0)
    wait_scat(1)
    plsc.subcore_barrier()
    pltpu.sync_copy(acc.at[pl.ds(rows0, NPT)], out_hbm.at[c, pl.ds(rows0, NPT)])


def _node_mlp_body(x_ref, p0_ref, p1_ref, wh1x_ref, wh1m_ref, wh1n_ref,
                   bh1_ref, wh2_ref, bh2_ref, out_ref):
    xb = x_ref[...]
    p0 = p0_ref[...]
    p1 = p1_ref[...]
    mh = p0[:, 0:H] + p1[:, 0:H]
    mvp = p0[:, H:H + 16] + p1[:, H:H + 16]
    nrm = jnp.sqrt(jnp.sum(mvp * mvp, axis=1, keepdims=True) + 1e-12)
    pre = (jnp.dot(xb, wh1x_ref[...], preferred_element_type=jnp.float32)
           + jnp.dot(mh, wh1m_ref[...], preferred_element_type=jnp.float32)
           + nrm * wh1n_ref[...] + bh1_ref[...])
    u = jnp.dot(_silu(pre), wh2_ref[...], preferred_element_type=jnp.float32)
    out_ref[...] = xb + u + bh2_ref[...]


def _node_mlp(x, p0, p1, wh1x, wh1m, wh1n, bh1, wh2, bh2):
    nb = 2000
    grid = N // nb
    return pl.pallas_call(
        _node_mlp_body,
        grid=(grid,),
        in_specs=[
            pl.BlockSpec((nb, D), lambda i: (i, 0)),
            pl.BlockSpec((nb, MW), lambda i: (i, 0)),
            pl.BlockSpec((nb, MW), lambda i: (i, 0)),
            pl.BlockSpec((D, H), lambda i: (0, 0)),
            pl.BlockSpec((H, H), lambda i: (0, 0)),
            pl.BlockSpec((1, H), lambda i: (0, 0)),
            pl.BlockSpec((1, H), lambda i: (0, 0)),
            pl.BlockSpec((H, D), lambda i: (0, 0)),
            pl.BlockSpec((1, D), lambda i: (0, 0)),
        ],
        out_specs=pl.BlockSpec((nb, D), lambda i: (i, 0)),
        out_shape=jax.ShapeDtypeStruct((N, D), jnp.float32),
    )(x, p0, p1, wh1x, wh1m, wh1n, bh1, wh2, bh2)


def kernel(x, pos, vel, edge_index, params):
    we1, be1 = params['phi_e'][0]
    we2, be2 = params['phi_e'][1]
    we3, be3 = params['phi_e'][2]
    wv1, bv1 = params['phi_v'][0]
    wv2, bv2 = params['phi_v'][1]
    wh1, bh1 = params['phi_h'][0]
    wh2, bh2 = params['phi_h'][1]

    wsrc = jnp.concatenate([we1[D:2 * D], wv1[D:2 * D]], axis=1)
    wdst = jnp.concatenate([we1[0:D], wv1[0:D]], axis=1)
    b1 = jnp.concatenate([be1, bv1])[None, :]
    r512 = jnp.concatenate([we1[2 * D], wv1[2 * D]])[None, :]
    r513 = jnp.concatenate([we1[2 * D + 1], wv1[2 * D + 1]])[None, :]
    m8 = jnp.concatenate(
        [r512, r513, jnp.zeros((6, PW), jnp.float32)], axis=0)
    wc = jnp.zeros((PW, PW), jnp.float32)
    wc = wc.at[0:H, 0:H].set(we2)
    wc = wc.at[H:2 * H, H:H + 1].set(wv2)
    bc = jnp.zeros((1, PW), jnp.float32)
    bc = bc.at[0, 0:H].set(be2)
    bc = bc.at[0, H].set(bv2[0])
    geom4 = jnp.concatenate([pos, vel], axis=1).reshape(-1)

    src = edge_index[0]
    dst = edge_index[1]

    ts, td = _node_proj(x, wsrc, wdst, b1)
    r, g = _edge_gather_kernel()(ts, td, src, dst, geom4)
    msg = _edge_mlp(r, g, m8, wc, bc, we3, be3[None, :])
    zeros = jnp.zeros((NP, MW), jnp.float32)
    partials = _edge_scatter_kernel()(msg, dst, g, zeros)

    out = _node_mlp(x, partials[0, :N], partials[1, :N],
                    wh1[0:D], wh1[D:D + H], wh1[D + H][None, :],
                    bh1[None, :], wh2, bh2[None, :])
    return out
